# baseline (speedup 1.0000x reference)
import functools

import jax
import jax.numpy as jnp
from jax.experimental import pallas as pl
from jax.experimental.pallas import tpu as pltpu

_N = 10000
_NP = 10240
_E = 320000
_HD = 128
_EBLK = 2000

_INTERPRET = False


def _update_kernel(do_embed, h_ref, embw_ref, embb_ref,
                   ah_ref, num_ref, den_ref, bng_ref, bnb_ref, h_out):
    h = h_ref[...]
    if do_embed:
        h = jnp.dot(h, embw_ref[...],
                    preferred_element_type=jnp.float32) + embb_ref[...]
    else:
        hnew = ah_ref[...] + num_ref[:_N, :] / (den_ref[:_N, :] + 1e-6)
        m = jnp.mean(hnew, axis=0, keepdims=True)
        v = jnp.mean((hnew - m) ** 2, axis=0, keepdims=True)
        hn = bng_ref[...] * (hnew - m) / jnp.sqrt(v + 1e-5) + bnb_ref[...]
        h = h + jnp.maximum(hn, 0.0)
    h_out[...] = h


def _update_call(do_embed, h, embw, embb, ah, num, den, bng, bnb):
    fn = functools.partial(_update_kernel, do_embed)
    return pl.pallas_call(
        fn,
        out_shape=jax.ShapeDtypeStruct((_N, _HD), jnp.float32),
        interpret=_INTERPRET,
    )(h, embw, embb, ah, num, den, bng, bnb)


def _tables_kernel(h_ref, aw_ref, ab_ref, bw_ref, bb_ref, dw_ref, db_ref,
                   ew_ref, eb_ref,
                   ah_out, s0_out, s1_out, t0_out, t1_out):
    h = h_ref[...]
    ah_out[...] = jnp.dot(h, aw_ref[...],
                          preferred_element_type=jnp.float32) + ab_ref[...]
    bh = jnp.dot(h, bw_ref[...],
                 preferred_element_type=jnp.float32) + bb_ref[...]
    dh = jnp.dot(h, dw_ref[...],
                 preferred_element_type=jnp.float32) + db_ref[...]
    eh = jnp.dot(h, ew_ref[...],
                 preferred_element_type=jnp.float32) + eb_ref[...]
    s0_out[...] = jnp.concatenate([dh[:, :64], bh[:, :64]], axis=1)
    s1_out[...] = jnp.concatenate([dh[:, 64:], bh[:, 64:]], axis=1)
    t0_out[...] = eh[:, :64]
    t1_out[...] = eh[:, 64:]


def _tables_call(h, aw, ab, bw, bb, dw, db, ew, eb):
    f32 = jnp.float32
    out_shape = [
        jax.ShapeDtypeStruct((_N, _HD), f32),
        jax.ShapeDtypeStruct((_N, _HD), f32),
        jax.ShapeDtypeStruct((_N, _HD), f32),
        jax.ShapeDtypeStruct((_N, 64), f32),
        jax.ShapeDtypeStruct((_N, 64), f32),
    ]
    return pl.pallas_call(
        _tables_kernel,
        out_shape=out_shape,
        interpret=_INTERPRET,
    )(h, aw, ab, bw, bb, dw, db, ew, eb)


def _edge_kernel(first, last, prev_ref, en0_ref, en1_ref, sc_ref, sh_ref,
                 w_ref, b_ref, cw_ref, cb_ref, *outs):
    if first:
        el = prev_ref[...]
    else:
        en = jnp.concatenate([en0_ref[...], en1_ref[...]], axis=1)
        en = en * sc_ref[...] + sh_ref[...]
        prev = prev_ref[...]
        if prev.shape[1] != _HD:
            prev = jnp.dot(prev, w_ref[...],
                           preferred_element_type=jnp.float32) + b_ref[...]
        el = prev + jnp.maximum(en, 0.0)
    ce = jnp.dot(el, cw_ref[...],
                 preferred_element_type=jnp.float32) + cb_ref[...]
    if last:
        ce0_out, ce1_out = outs
    else:
        ce0_out, ce1_out, el_out = outs
        el_out[...] = el
    ce0_out[...] = ce[:, :64]
    ce1_out[...] = ce[:, 64:]


def _edge_call(first, last, prev, en0, en1, scale, shift, w, b, cw, cb):
    f32 = jnp.float32
    nblk = _E // _EBLK
    pd = prev.shape[1]
    zero2 = lambda i: (0, 0)
    row = lambda i: (i, 0)
    in_specs = [
        pl.BlockSpec((_EBLK, pd), row),
        pl.BlockSpec((_EBLK, 64), row),
        pl.BlockSpec((_EBLK, 64), row),
        pl.BlockSpec((1, _HD), zero2),
        pl.BlockSpec((1, _HD), zero2),
        pl.BlockSpec(w.shape, zero2),
        pl.BlockSpec((1, _HD), zero2),
        pl.BlockSpec(cw.shape, zero2),
        pl.BlockSpec((1, _HD), zero2),
    ]
    out_shape = [
        jax.ShapeDtypeStruct((_E, 64), f32),
        jax.ShapeDtypeStruct((_E, 64), f32),
    ]
    out_specs = [
        pl.BlockSpec((_EBLK, 64), row),
        pl.BlockSpec((_EBLK, 64), row),
    ]
    if not last:
        out_shape.append(jax.ShapeDtypeStruct((_E, _HD), f32))
        out_specs.append(pl.BlockSpec((_EBLK, _HD), row))
    fn = functools.partial(_edge_kernel, first, last)
    return pl.pallas_call(
        fn,
        grid=(nblk,),
        in_specs=in_specs,
        out_specs=out_specs,
        out_shape=out_shape,
        interpret=_INTERPRET,
    )(prev, en0, en1, scale, shift, w, b, cw, cb)


def _edge_pass(ce0, ce1, s0, s1, t0, t1, src, dst, last):
    ce = jnp.concatenate([ce0, ce1], axis=1)
    dh = jnp.concatenate([s0[:, :64], s1[:, :64]], axis=1)
    bh = jnp.concatenate([s0[:, 64:], s1[:, 64:]], axis=1)
    eh = jnp.concatenate([t0, t1], axis=1)
    en = ce + dh[src] + eh[dst]
    sg = jax.nn.sigmoid(en)
    num = jax.ops.segment_sum(sg * bh[src], dst, num_segments=_N)
    den = jax.ops.segment_sum(sg, dst, num_segments=_N)
    nump = jnp.zeros((_NP, _HD), jnp.float32).at[:_N].set(num)
    denp = jnp.zeros((_NP, _HD), jnp.float32).at[:_N].set(den)
    if last:
        return None, None, nump, denp, None, None
    s = jnp.sum(en, axis=0)
    s2 = jnp.sum(en * en, axis=0)
    return en[:, :64], en[:, 64:], nump, denp, s, s2


def kernel(h, e, edge_index, params):
    f32 = jnp.float32
    src = edge_index[0]
    dst = edge_index[1]
    p = params
    row = lambda x: x.reshape(1, -1).astype(f32)
    zeros_r = jnp.zeros((1, _HD), f32)
    zeros_w = jnp.zeros((_HD, _HD), f32)
    zeros_d = jnp.zeros((8, _HD), f32)

    h0 = _update_call(True, h, p['emb_h_w'], row(p['emb_h_b']),
                      zeros_d, zeros_d, zeros_d, zeros_r, zeros_r)
    ah, s0, s1, t0, t1 = _tables_call(
        h0,
        p['A_w'][0], row(p['A_b'][0]), p['B_w'][0], row(p['B_b'][0]),
        p['D_w'][0], row(p['D_b'][0]), p['E_w'][0], row(p['E_b'][0]))

    w_fold = (p['emb_e_w'] @ p['C_w'][0]).astype(f32)
    b_fold = row(p['emb_e_b'] @ p['C_w'][0] + p['C_b'][0])
    ze = jnp.zeros((_E, 64), f32)
    ce0, ce1 = _edge_call(True, True, e, ze, ze, zeros_r, zeros_r,
                          zeros_w, zeros_r, w_fold, b_fold)

    hcur = h0
    ahcur = ah
    eprev = e
    en0 = en1 = esum = esum2 = None
    for i in range(4):
        last = i == 3
        if i > 0:
            m = (esum / _E).reshape(1, -1)
            v = (esum2 / _E).reshape(1, -1) - m * m
            scale = row(p['bn_e_g'][i - 1]) / jnp.sqrt(v + 1e-5)
            shift = row(p['bn_e_b'][i - 1]) - m * scale
            if i == 1:
                w = p['emb_e_w'].astype(f32)
                b = row(p['emb_e_b'])
            else:
                w = zeros_w
                b = zeros_r
            outs = _edge_call(False, last, eprev, en0, en1, scale, shift,
                              w, b, p['C_w'][i], row(p['C_b'][i]))
            if last:
                ce0, ce1 = outs
            else:
                ce0, ce1, eprev = outs
        en0, en1, num, den, esum, esum2 = _edge_pass(
            ce0, ce1, s0, s1, t0, t1, src, dst, last)
        hcur = _update_call(False, hcur, zeros_w, zeros_r, ahcur, num, den,
                            row(p['bn_h_g'][i]), row(p['bn_h_b'][i]))
        if last:
            return hcur
        ah, s0, s1, t0, t1 = _tables_call(
            hcur,
            p['A_w'][i + 1], row(p['A_b'][i + 1]),
            p['B_w'][i + 1], row(p['B_b'][i + 1]),
            p['D_w'][i + 1], row(p['D_b'][i + 1]),
            p['E_w'][i + 1], row(p['E_b'][i + 1]))
        ahcur = ah


# speedup vs baseline: 2.7972x; 2.7972x over previous
import functools

import jax
import jax.numpy as jnp
from jax import lax
from jax.experimental import pallas as pl
from jax.experimental.pallas import tpu as pltpu
from jax.experimental.pallas import tpu_sc as plsc

_N = 10000
_NP = 10240
_E = 320000
_HD = 128
_EBLK = 2000

_INTERPRET = False


def _update_kernel(do_embed, h_ref, embw_ref, embb_ref,
                   ah_ref, nd_ref, bng_ref, bnb_ref, h_out):
    h = h_ref[...]
    if do_embed:
        h = jnp.dot(h, embw_ref[...],
                    preferred_element_type=jnp.float32) + embb_ref[...]
    else:
        num = jnp.concatenate([nd_ref[0, :_N, :64], nd_ref[1, :_N, :64]],
                              axis=1)
        den = jnp.concatenate([nd_ref[0, :_N, 64:], nd_ref[1, :_N, 64:]],
                              axis=1)
        hnew = ah_ref[...] + num / (den + 1e-6)
        m = jnp.mean(hnew, axis=0, keepdims=True)
        v = jnp.mean((hnew - m) ** 2, axis=0, keepdims=True)
        hn = bng_ref[...] * (hnew - m) / jnp.sqrt(v + 1e-5) + bnb_ref[...]
        h = h + jnp.maximum(hn, 0.0)
    h_out[...] = h


def _update_call(do_embed, h, embw, embb, ah, nd, bng, bnb):
    fn = functools.partial(_update_kernel, do_embed)
    return pl.pallas_call(
        fn,
        out_shape=jax.ShapeDtypeStruct((_N, _HD), jnp.float32),
        interpret=_INTERPRET,
    )(h, embw, embb, ah, nd, bng, bnb)


def _tables_kernel(h_ref, aw_ref, ab_ref, bw_ref, bb_ref, dw_ref, db_ref,
                   ew_ref, eb_ref,
                   ah_out, s0_out, t0_out):
    h = h_ref[...]
    ah_out[...] = jnp.dot(h, aw_ref[...],
                          preferred_element_type=jnp.float32) + ab_ref[...]
    bh = jnp.dot(h, bw_ref[...],
                 preferred_element_type=jnp.float32) + bb_ref[...]
    dh = jnp.dot(h, dw_ref[...],
                 preferred_element_type=jnp.float32) + db_ref[...]
    eh = jnp.dot(h, ew_ref[...],
                 preferred_element_type=jnp.float32) + eb_ref[...]
    s0_out[:_N, :] = jnp.concatenate([dh[:, :64], bh[:, :64]], axis=1)
    s0_out[_N:, :] = jnp.concatenate([dh[:, 64:], bh[:, 64:]], axis=1)
    t0_out[:_N, :] = eh
    t0_out[_N:, :] = jnp.concatenate([eh[:, 64:], eh[:, :64]], axis=1)


def _tables_call(h, aw, ab, bw, bb, dw, db, ew, eb):
    f32 = jnp.float32
    out_shape = [
        jax.ShapeDtypeStruct((_N, _HD), f32),
        jax.ShapeDtypeStruct((2 * _N, _HD), f32),
        jax.ShapeDtypeStruct((2 * _N, _HD), f32),
    ]
    return pl.pallas_call(
        _tables_kernel,
        out_shape=out_shape,
        interpret=_INTERPRET,
    )(h, aw, ab, bw, bb, dw, db, ew, eb)


def _edge_kernel(first, last, prev_ref, en_ref, sc_ref, sh_ref,
                 w_ref, b_ref, cw_ref, cb_ref, *outs):
    if first:
        el = prev_ref[...]
    else:
        en = jnp.concatenate([en_ref[0], en_ref[1]], axis=1)
        en = en * sc_ref[...] + sh_ref[...]
        prev = prev_ref[...]
        if prev.shape[1] != _HD:
            prev = jnp.dot(prev, w_ref[...],
                           preferred_element_type=jnp.float32) + b_ref[...]
        el = prev + jnp.maximum(en, 0.0)
    ce = jnp.dot(el, cw_ref[...],
                 preferred_element_type=jnp.float32) + cb_ref[...]
    if last:
        ce_out, = outs
    else:
        ce_out, el_out = outs
        el_out[...] = el
    ce_out[0] = ce[:, :64]
    ce_out[1] = ce[:, 64:]


def _edge_call(first, last, prev, en, scale, shift, w, b, cw, cb):
    f32 = jnp.float32
    nblk = _E // _EBLK
    pd = prev.shape[1]
    zero2 = lambda i: (0, 0)
    row = lambda i: (i, 0)
    half = lambda i: (0, i, 0)
    in_specs = [
        pl.BlockSpec((_EBLK, pd), row),
        pl.BlockSpec((2, _EBLK, 64), half),
        pl.BlockSpec((1, _HD), zero2),
        pl.BlockSpec((1, _HD), zero2),
        pl.BlockSpec(w.shape, zero2),
        pl.BlockSpec((1, _HD), zero2),
        pl.BlockSpec(cw.shape, zero2),
        pl.BlockSpec((1, _HD), zero2),
    ]
    out_shape = [jax.ShapeDtypeStruct((2, _E, 64), f32)]
    out_specs = [pl.BlockSpec((2, _EBLK, 64), half)]
    if not last:
        out_shape.append(jax.ShapeDtypeStruct((_E, _HD), f32))
        out_specs.append(pl.BlockSpec((_EBLK, _HD), row))
    fn = functools.partial(_edge_kernel, first, last)
    return pl.pallas_call(
        fn,
        grid=(nblk,),
        in_specs=in_specs,
        out_specs=out_specs,
        out_shape=out_shape,
        interpret=_INTERPRET,
    )(prev, en, scale, shift, w, b, cw, cb)


_CH = 80
_EPT = _E // 16
_NIT = _EPT // _CH
_ZR = 64


def _sc_edge_kernel(last, ce_hbm, stab_hbm, ttab_hbm, src_hbm, dst_hbm,
                    *rest):
    if last:
        (nd_out, bn_out,
         srcv, dstv, dstadj, srows, trows, cev, msgsg,
         accb, ndacc, sem1, sem2) = rest
        en_out = None
    else:
        (en_out, nd_out, bn_out,
         srcv, dstv, dstadj, srows, trows, cev, msgsg,
         accb, ndacc, sem1, sem2) = rest
    c = lax.axis_index("c")
    s = lax.axis_index("s")
    zero = jnp.zeros((16,), jnp.float32)

    def zb_body(k, _):
        msgsg[k // 8, pl.ds(16 * (k % 8), 16)] = zero
        return 0
    lax.fori_loop(0, _CH * 8, zb_body, 0)

    def zacc_body(m, _):
        base = s * (_NP // 16) + m * _CH
        pltpu.sync_copy(msgsg, ndacc.at[pl.ds(base, _CH)])
        return 0
    lax.fori_loop(0, _NP // 16 // _CH, zacc_body, 0)
    plsc.subcore_barrier()

    coff = c * _N

    def chunk(i, acc):
        base = s * _EPT + i * _CH
        pltpu.sync_copy(src_hbm.at[pl.ds(base, _CH)], srcv)
        pltpu.sync_copy(dst_hbm.at[pl.ds(base, _CH)], dstv)

        def adj_body(k, _):
            sl = pl.ds(16 * k, 16)
            srcv[sl] = srcv[sl] + coff
            dstadj[sl] = dstv[sl] + coff
            return 0
        lax.fori_loop(0, _CH // 16, adj_body, 0)

        cp1 = pltpu.async_copy(stab_hbm.at[srcv], srows, sem1)
        cp2 = pltpu.async_copy(ttab_hbm.at[dstadj], trows, sem2)
        pltpu.sync_copy(ce_hbm.at[c, pl.ds(base, _CH)], cev)
        cp1.wait()
        cp2.wait()

        def ej(j, a):
            a = list(a)
            for q in range(4):
                sl = pl.ds(16 * q, 16)
                en = cev[j, sl] + srows[j, sl] + trows[j, sl]
                sg = 1.0 / (1.0 + jnp.exp(-en))
                if not last:
                    cev[j, sl] = en
                msgsg[j, sl] = sg * srows[j, pl.ds(64 + 16 * q, 16)]
                msgsg[j, pl.ds(64 + 16 * q, 16)] = sg
                a[q] = a[q] + en
                a[4 + q] = a[4 + q] + en * en
            return tuple(a)
        acc = lax.fori_loop(0, _CH, ej, acc)

        if not last:
            pltpu.sync_copy(cev, en_out.at[c, pl.ds(base, _CH)])
        pltpu.sync_copy(msgsg, ndacc.at[dstv], add=True)
        return acc

    acc0 = (zero,) * 8
    acc = lax.fori_loop(0, _NIT, chunk, acc0)

    for q in range(4):
        accb[pl.ds(16 * q, 16)] = acc[q]
        accb[pl.ds(64 + 16 * q, 16)] = acc[4 + q]
    pltpu.sync_copy(accb, bn_out.at[c * 16 + s])

    plsc.subcore_barrier()
    rows = _NP // 16
    pltpu.sync_copy(ndacc.at[pl.ds(s * rows, rows)],
                    nd_out.at[c, pl.ds(s * rows, rows)])


def _edge_pass(ce, stab, ttab, src, dst, last):
    f32 = jnp.float32
    mesh = plsc.VectorSubcoreMesh(core_axis_name="c", subcore_axis_name="s",
                                  num_cores=2, num_subcores=16)
    out_type = []
    if not last:
        out_type.append(jax.ShapeDtypeStruct((2, _E, 64), f32))
    out_type += [
        jax.ShapeDtypeStruct((2, _NP, _HD), f32),
        jax.ShapeDtypeStruct((32, _HD), f32),
    ]
    scratch = [
        pltpu.VMEM((_CH,), jnp.int32),
        pltpu.VMEM((_CH,), jnp.int32),
        pltpu.VMEM((_CH,), jnp.int32),
        pltpu.VMEM((_CH, _HD), f32),
        pltpu.VMEM((_CH, _HD), f32),
        pltpu.VMEM((_CH, 64), f32),
        pltpu.VMEM((_CH, _HD), f32),
        pltpu.VMEM((_HD,), f32),
        pltpu.VMEM_SHARED((_NP, _HD), f32),
        pltpu.SemaphoreType.DMA,
        pltpu.SemaphoreType.DMA,
    ]
    fn = pl.kernel(functools.partial(_sc_edge_kernel, last),
                   out_type=out_type, mesh=mesh, scratch_types=scratch,
                   interpret=_INTERPRET)
    outs = fn(ce, stab, ttab, src, dst)
    if last:
        nd, bn = outs
        return None, nd, None, None
    en, nd, bn = outs
    esum = jnp.concatenate([jnp.sum(bn[:16, :64], 0), jnp.sum(bn[16:, :64], 0)])
    esum2 = jnp.concatenate([jnp.sum(bn[:16, 64:], 0), jnp.sum(bn[16:, 64:], 0)])
    return en, nd, esum, esum2


def kernel(h, e, edge_index, params):
    f32 = jnp.float32
    src = edge_index[0]
    dst = edge_index[1]
    p = params
    row = lambda x: x.reshape(1, -1).astype(f32)
    zeros_r = jnp.zeros((1, _HD), f32)
    zeros_w = jnp.zeros((_HD, _HD), f32)
    zeros_d = jnp.zeros((2, 8, _HD), f32)

    h0 = _update_call(True, h, p['emb_h_w'], row(p['emb_h_b']),
                      jnp.zeros((8, _HD), f32), zeros_d,
                      zeros_r, zeros_r)
    ah, stab, ttab = _tables_call(
        h0,
        p['A_w'][0], row(p['A_b'][0]), p['B_w'][0], row(p['B_b'][0]),
        p['D_w'][0], row(p['D_b'][0]), p['E_w'][0], row(p['E_b'][0]))

    w_fold = (p['emb_e_w'] @ p['C_w'][0]).astype(f32)
    b_fold = row(p['emb_e_b'] @ p['C_w'][0] + p['C_b'][0])
    ze = jnp.zeros((2, _E, 64), f32)
    ce, = _edge_call(True, True, e, ze, zeros_r, zeros_r,
                     zeros_w, zeros_r, w_fold, b_fold)

    hcur = h0
    ahcur = ah
    eprev = e
    en = esum = esum2 = None
    for i in range(4):
        last = i == 3
        if i > 0:
            m = (esum / _E).reshape(1, -1)
            v = (esum2 / _E).reshape(1, -1) - m * m
            scale = row(p['bn_e_g'][i - 1]) / jnp.sqrt(v + 1e-5)
            shift = row(p['bn_e_b'][i - 1]) - m * scale
            if i == 1:
                w = p['emb_e_w'].astype(f32)
                b = row(p['emb_e_b'])
            else:
                w = zeros_w
                b = zeros_r
            outs = _edge_call(False, last, eprev, en, scale, shift,
                              w, b, p['C_w'][i], row(p['C_b'][i]))
            if last:
                ce, = outs
            else:
                ce, eprev = outs
        en, nd, esum, esum2 = _edge_pass(ce, stab, ttab, src, dst, last)
        hcur = _update_call(False, hcur, zeros_w, zeros_r, ahcur, nd,
                            row(p['bn_h_g'][i]), row(p['bn_h_b'][i]))
        if last:
            return hcur
        ah, stab, ttab = _tables_call(
            hcur,
            p['A_w'][i + 1], row(p['A_b'][i + 1]),
            p['B_w'][i + 1], row(p['B_b'][i + 1]),
            p['D_w'][i + 1], row(p['D_b'][i + 1]),
            p['E_w'][i + 1], row(p['E_b'][i + 1]))
        ahcur = ah
